# baseline (speedup 1.0000x reference)
import functools

import jax
import jax.numpy as jnp
from jax.experimental import pallas as pl
from jax.experimental.pallas import tpu as pltpu

_NEG = -1e30
_LOG2E = 1.4426950408889634
_LN2 = 0.6931471805599453


def _loss_body(hb_ref, w_ref, b_ref, t_ref, h_ref, out_ref,
               m_ref, s_ref, z_ref, *, v_chunk, n_chunks):
    i = pl.program_id(0)

    @pl.when(i == 0)
    def _init():
        m_ref[:] = jnp.full(m_ref.shape, _NEG, jnp.float32)
        s_ref[:] = jnp.zeros(s_ref.shape, jnp.float32)
        z_ref[:] = jnp.zeros(z_ref.shape, jnp.float32)

    w = w_ref[:]
    chunk = jax.lax.dot_general(
        hb_ref[:], w.astype(jnp.bfloat16),
        (((1,), (1,)), ((), ())),
        preferred_element_type=jnp.float32) + b_ref[0]

    n = chunk.shape[0]
    col = i * v_chunk + jax.lax.broadcasted_iota(jnp.int32, (n, v_chunk), 1)

    t = t_ref[:]
    z_part = jnp.sum(jnp.where(col == t, chunk, 0.0), axis=1, keepdims=True)
    z_ref[:] = z_ref[:] + z_part

    cmax = jnp.max(chunk, axis=1, keepdims=True)
    m_old = m_ref[:]
    m_new = jnp.maximum(m_old, cmax)
    s_ref[:] = (s_ref[:] * jnp.exp2(m_old - m_new)
                + jnp.sum(jnp.exp2(chunk - m_new), axis=1, keepdims=True))
    m_ref[:] = m_new

    @pl.when(i == n_chunks - 1)
    def _final():
        lse2 = m_ref[:] + jnp.log2(s_ref[:])
        wgt = (t != 0).astype(jnp.float32)
        out_ref[:] = _LN2 * jnp.sum(wgt * (lse2 - z_ref[:]), keepdims=True)


def _nmt_loss(hb, t2, w_mat, b3, h, *, v_chunk=2000, interpret=False):
    n, d = hb.shape
    v = w_mat.shape[0]
    n_chunks = pl.cdiv(v, v_chunk)
    assert v % v_chunk == 0

    body = functools.partial(_loss_body, v_chunk=v_chunk, n_chunks=n_chunks)
    out = pl.pallas_call(
        body,
        grid=(n_chunks,),
        in_specs=[
            pl.BlockSpec((n, d), lambda i: (0, 0)),
            pl.BlockSpec((v_chunk, d), lambda i: (i, 0)),
            pl.BlockSpec((1, 1, v_chunk), lambda i: (i, 0, 0)),
            pl.BlockSpec((n, 1), lambda i: (0, 0)),
            pl.BlockSpec((n, d), lambda i: (0, 0)),
        ],
        out_specs=pl.BlockSpec((1, 1), lambda i: (0, 0)),
        out_shape=jax.ShapeDtypeStruct((1, 1), jnp.float32),
        scratch_shapes=[
            pltpu.VMEM((n, 1), jnp.float32),
            pltpu.VMEM((n, 1), jnp.float32),
            pltpu.VMEM((n, 1), jnp.float32),
        ],
        compiler_params=pltpu.CompilerParams(
            dimension_semantics=("arbitrary",)),
        interpret=interpret,
    )(hb, w_mat, b3, t2, h)
    return out[0, 0]


def kernel(hiddens, targets, W, b):
    t, bsz, d = hiddens.shape
    h = hiddens.reshape(t * bsz, d)
    hb = (h * _LOG2E).astype(jnp.bfloat16)
    t2 = targets.reshape(t * bsz, 1).astype(jnp.int32)
    v_chunk = 2000
    b3 = (b * _LOG2E).reshape(-1, 1, v_chunk)
    return _nmt_loss(hb, t2, W, b3, h, v_chunk=v_chunk)
